# baseline (speedup 1.0000x reference)
import numpy as np
import jax
import jax.numpy as jnp
from jax.experimental import pallas as pl
from jax.experimental.pallas import tpu as pltpu

_EPS = 1e-5

_CIN, _HIN, _WIN = 3, 32, 32
_KS = 5
_CO1, _HC1, _WC1, _HP1, _WP1 = 6, 28, 28, 14, 14
_CO2, _HC2, _WC2, _HP2, _WP2 = 16, 10, 10, 5, 5
_LIN = _WIN * _CIN
_LC1 = _WC1 * _CO1
_LQ1 = _WP1 * _CO1
_LC2 = _WC2 * _CO2
_LQ2 = _WP2 * _CO2

_CP = pltpu.CompilerParams(
    dimension_semantics=("parallel",),
    vmem_limit_bytes=64 * 1024 * 1024,
)


def _stage1(x_ref, w_ref, b_ref, eo_ref, pmax_ref, pmin_ref, st_ref):
    bt = x_ref.shape[0]
    rows = bt * _HC1
    x = x_ref[...]
    acc = jnp.zeros((rows, _LC1), jnp.float32)
    for kh in range(_KS):
        xs = x[:, kh:kh + _HC1, :].reshape(rows, _LIN)
        acc = acc + jnp.dot(xs, w_ref[kh], preferred_element_type=jnp.float32)
    y = acc + b_ref[...]
    s = jnp.sum(y, axis=0, keepdims=True)
    q = jnp.sum((y - s * (1.0 / rows)) ** 2, axis=0, keepdims=True)
    st_ref[...] = jnp.concatenate([s, q], axis=0).reshape(1, 2, _LC1)
    yb = y.astype(jnp.bfloat16)
    e = jnp.dot(yb, eo_ref[0], preferred_element_type=jnp.float32)
    o = jnp.dot(yb, eo_ref[1], preferred_element_type=jnp.float32)
    wmax = jnp.maximum(e, o).reshape(bt, _HP1, 2, _LQ1)
    wmin = jnp.minimum(e, o).reshape(bt, _HP1, 2, _LQ1)
    pmax_ref[...] = jnp.maximum(wmax[:, :, 0], wmax[:, :, 1]).reshape(
        bt * _HP1, _LQ1).astype(jnp.bfloat16)
    pmin_ref[...] = jnp.minimum(wmin[:, :, 0], wmin[:, :, 1]).reshape(
        bt * _HP1, _LQ1).astype(jnp.bfloat16)


def _stage2(pmax_ref, pmin_ref, sc_ref, sh_ref, w_ref, b_ref, eo_ref,
            qmax_ref, qmin_ref, st_ref):
    bt = pmax_ref.shape[0] // _HP1
    sc = sc_ref[...]
    xsel = jnp.where(sc >= 0.0, pmax_ref[...].astype(jnp.float32),
                     pmin_ref[...].astype(jnp.float32))
    x2 = jnp.maximum(xsel * sc + sh_ref[...], 0.0).astype(jnp.bfloat16)
    x2 = x2.reshape(bt, _HP1, _LQ1)
    rows = bt * _HC2
    acc = jnp.zeros((rows, _LC2), jnp.float32)
    for kh in range(_KS):
        xs = x2[:, kh:kh + _HC2, :].reshape(rows, _LQ1)
        acc = acc + jnp.dot(xs, w_ref[kh], preferred_element_type=jnp.float32)
    y = acc + b_ref[...]
    s = jnp.sum(y, axis=0, keepdims=True)
    q = jnp.sum((y - s * (1.0 / rows)) ** 2, axis=0, keepdims=True)
    st_ref[...] = jnp.concatenate([s, q], axis=0).reshape(1, 2, _LC2)
    yb = y.astype(jnp.bfloat16)
    e = jnp.dot(yb, eo_ref[0], preferred_element_type=jnp.float32)
    o = jnp.dot(yb, eo_ref[1], preferred_element_type=jnp.float32)
    wmax = jnp.maximum(e, o).reshape(bt, _HP2, 2, _LQ2)
    wmin = jnp.minimum(e, o).reshape(bt, _HP2, 2, _LQ2)
    qmax_ref[...] = jnp.maximum(wmax[:, :, 0], wmax[:, :, 1]).reshape(
        bt * _HP2, _LQ2).astype(jnp.bfloat16)
    qmin_ref[...] = jnp.minimum(wmin[:, :, 0], wmin[:, :, 1]).reshape(
        bt * _HP2, _LQ2).astype(jnp.bfloat16)


def _stage3(qmax_ref, qmin_ref, sc_ref, sh_ref, w3_ref, b3_ref, fold_ref,
            o_ref):
    bt = qmax_ref.shape[0] // _HP2
    nc = fold_ref.shape[1]
    sc = sc_ref[...]
    xsel = jnp.where(sc >= 0.0, qmax_ref[...].astype(jnp.float32),
                     qmin_ref[...].astype(jnp.float32))
    xp = jnp.maximum(xsel * sc + sh_ref[...], 0.0).astype(jnp.bfloat16)
    z = jnp.maximum(
        jnp.dot(xp, w3_ref[...], preferred_element_type=jnp.float32)
        + b3_ref[...], 0.0)
    zs = jnp.sum(z.reshape(bt, _HP2, _WP2 * nc), axis=1)
    o_ref[...] = jnp.dot(zs, fold_ref[...],
                         preferred_element_type=jnp.float32) * (1.0 / (_HP2 * _WP2))


def _banded(wt, wout):
    co, ci, kh, kw = wt.shape
    win = wout + kw - 1
    band = np.zeros((kw, win, wout), np.float32)
    for k in range(kw):
        band[k, np.arange(wout) + k, np.arange(wout)] = 1.0
    wp = jnp.transpose(wt, (2, 3, 1, 0)).astype(jnp.float32)
    wb = jnp.einsum("wrj,hwio->hrijo", jnp.asarray(band), wp)
    return wb.reshape(kh, win * ci, wout * co)


def _even_odd(pairs, c):
    li, lo = 2 * pairs * c, pairs * c
    icol, ich = np.arange(li) // c, np.arange(li) % c
    ocol, och = np.arange(lo) // c, np.arange(lo) % c
    ch_ok = ich[:, None] == och[None, :]
    ev = ((icol[:, None] == 2 * ocol[None, :]) & ch_ok)
    od = ((icol[:, None] == 2 * ocol[None, :] + 1) & ch_ok)
    return jnp.asarray(np.stack([ev, od]).astype(np.float32))


def _bn_fold(st, rows_tile, wout, wtile, gamma, beta):
    nt, _, lanes = st.shape
    c = gamma.shape[0]
    s, q = st[:, 0, :], st[:, 1, :]
    n = nt * rows_tile * wout
    sum_c = jnp.sum(s.reshape(nt, wout, c), axis=(0, 1))
    mean_c = sum_c / n
    m2 = q + rows_tile * (s * (1.0 / rows_tile) - jnp.tile(mean_c, wout)[None, :]) ** 2
    var_c = jnp.sum(m2.reshape(nt, wout, c), axis=(0, 1)) / n
    sc_c = gamma * jax.lax.rsqrt(var_c + _EPS)
    sh_c = beta - mean_c * sc_c
    return (jnp.tile(sc_c, wtile).reshape(1, wtile * c),
            jnp.tile(sh_c, wtile).reshape(1, wtile * c))


def _tile(b):
    for t in (128, 64, 32, 16, 8):
        if b % t == 0:
            return t
    return b


def kernel(x, w1, b1, g1, be1, w2, b2, g2, be2, w3, b3):
    x = x.reshape(-1, _CIN, _HIN, _WIN)
    b = x.shape[0]
    nc = w3.shape[0]
    bt = _tile(b)
    nt = b // bt

    xm = jnp.transpose(x.astype(jnp.bfloat16), (0, 2, 3, 1)).reshape(b, _HIN, _LIN)

    wb1 = _banded(w1, _WC1).astype(jnp.bfloat16)
    wb2 = _banded(w2, _WC2).astype(jnp.bfloat16)
    b1t = jnp.tile(b1, _WC1).reshape(1, _LC1).astype(jnp.float32)
    b2t = jnp.tile(b2, _WC2).reshape(1, _LC2).astype(jnp.float32)
    eo1 = _even_odd(_WP1, _CO1).astype(jnp.bfloat16)
    eo2 = _even_odd(_WP2, _CO2).astype(jnp.bfloat16)
    w3m = jnp.transpose(w3.reshape(nc, _CO2)).astype(jnp.float32)
    w3b = jnp.einsum("gh,io->giho", jnp.eye(_WP2, dtype=jnp.float32),
                     w3m).reshape(_WP2 * _CO2, _WP2 * nc).astype(jnp.bfloat16)
    b3t = jnp.tile(b3, _WP2).reshape(1, _WP2 * nc).astype(jnp.float32)
    fold = jnp.asarray(np.tile(np.eye(nc, dtype=np.float32), (_WP2, 1)))

    pmax1, pmin1, st1 = pl.pallas_call(
        _stage1,
        grid=(nt,),
        in_specs=[pl.BlockSpec((bt, _HIN, _LIN), lambda t: (t, 0, 0)),
                  pl.BlockSpec((_KS, _LIN, _LC1), lambda t: (0, 0, 0)),
                  pl.BlockSpec((1, _LC1), lambda t: (0, 0)),
                  pl.BlockSpec((2, _LC1, _LQ1), lambda t: (0, 0, 0))],
        out_specs=[pl.BlockSpec((bt * _HP1, _LQ1), lambda t: (t, 0)),
                   pl.BlockSpec((bt * _HP1, _LQ1), lambda t: (t, 0)),
                   pl.BlockSpec((1, 2, _LC1), lambda t: (t, 0, 0))],
        out_shape=[jax.ShapeDtypeStruct((b * _HP1, _LQ1), jnp.bfloat16),
                   jax.ShapeDtypeStruct((b * _HP1, _LQ1), jnp.bfloat16),
                   jax.ShapeDtypeStruct((nt, 2, _LC1), jnp.float32)],
        compiler_params=_CP,
    )(xm, wb1, b1t, eo1)
    sc1, sh1 = _bn_fold(st1, bt * _HC1, _WC1, _WP1, g1, be1)

    qmax2, qmin2, st2 = pl.pallas_call(
        _stage2,
        grid=(nt,),
        in_specs=[pl.BlockSpec((bt * _HP1, _LQ1), lambda t: (t, 0)),
                  pl.BlockSpec((bt * _HP1, _LQ1), lambda t: (t, 0)),
                  pl.BlockSpec((1, _LQ1), lambda t: (0, 0)),
                  pl.BlockSpec((1, _LQ1), lambda t: (0, 0)),
                  pl.BlockSpec((_KS, _LQ1, _LC2), lambda t: (0, 0, 0)),
                  pl.BlockSpec((1, _LC2), lambda t: (0, 0)),
                  pl.BlockSpec((2, _LC2, _LQ2), lambda t: (0, 0, 0))],
        out_specs=[pl.BlockSpec((bt * _HP2, _LQ2), lambda t: (t, 0)),
                   pl.BlockSpec((bt * _HP2, _LQ2), lambda t: (t, 0)),
                   pl.BlockSpec((1, 2, _LC2), lambda t: (t, 0, 0))],
        out_shape=[jax.ShapeDtypeStruct((b * _HP2, _LQ2), jnp.bfloat16),
                   jax.ShapeDtypeStruct((b * _HP2, _LQ2), jnp.bfloat16),
                   jax.ShapeDtypeStruct((nt, 2, _LC2), jnp.float32)],
        compiler_params=_CP,
    )(pmax1, pmin1, sc1, sh1, wb2, b2t, eo2)
    sc2, sh2 = _bn_fold(st2, bt * _HC2, _WC2, _WP2, g2, be2)

    out = pl.pallas_call(
        _stage3,
        grid=(nt,),
        in_specs=[pl.BlockSpec((bt * _HP2, _LQ2), lambda t: (t, 0)),
                  pl.BlockSpec((bt * _HP2, _LQ2), lambda t: (t, 0)),
                  pl.BlockSpec((1, _LQ2), lambda t: (0, 0)),
                  pl.BlockSpec((1, _LQ2), lambda t: (0, 0)),
                  pl.BlockSpec((_LQ2, _WP2 * nc), lambda t: (0, 0)),
                  pl.BlockSpec((1, _WP2 * nc), lambda t: (0, 0)),
                  pl.BlockSpec((_WP2 * nc, nc), lambda t: (0, 0))],
        out_specs=pl.BlockSpec((bt, nc), lambda t: (t, 0)),
        out_shape=jax.ShapeDtypeStruct((b, nc), jnp.float32),
        compiler_params=_CP,
    )(qmax2, qmin2, sc2, sh2, w3b, b3t, fold)
    return out
